# baseline (speedup 1.0000x reference)
import functools

import jax
import jax.numpy as jnp
from jax import lax
from jax.experimental import pallas as pl
from jax.experimental.pallas import tpu as pltpu
from jax.experimental.pallas import tpu_sc as plsc

N_ATOMS = 10000
N_BONDS = 320000
N_TYPES = 20

_NC = 2
_NS = 16
_NW = _NC * _NS
_L = 16
_E_PER = N_BONDS // _NW
_CHUNKS = _E_PER // _L


def _sqrt16(s):
    i = lax.bitcast_convert_type(s, jnp.int32)
    i = jnp.int32(0x5F3759DF) - lax.shift_right_logical(i, 1)
    r = lax.bitcast_convert_type(i, jnp.float32)
    half = s * jnp.float32(0.5)
    for _ in range(3):
        r = r * (jnp.float32(1.5) - half * r * r)
    return s * r


def _body(posx_h, posy_h, posz_h, typ_h, x0_h, k_h, src_h, dst_h, out_h,
          posx_v, posy_v, posz_v, typ_v, x0_v, k_v, src_v, dst_v, out_v, sem):
    wid = lax.axis_index("s") * _NC + lax.axis_index("c")
    base = wid * _E_PER

    copies = [
        pltpu.make_async_copy(posx_h, posx_v, sem),
        pltpu.make_async_copy(posy_h, posy_v, sem),
        pltpu.make_async_copy(posz_h, posz_v, sem),
        pltpu.make_async_copy(typ_h, typ_v, sem),
        pltpu.make_async_copy(x0_h, x0_v, sem),
        pltpu.make_async_copy(k_h, k_v, sem),
        pltpu.make_async_copy(src_h.at[pl.ds(base, _E_PER)], src_v, sem),
        pltpu.make_async_copy(dst_h.at[pl.ds(base, _E_PER)], dst_v, sem),
    ]
    for cp in copies:
        cp.start()
    for cp in copies:
        cp.wait()

    def chunk(c, carry):
        off = c * _L
        i = src_v[pl.ds(off, _L)]
        j = dst_v[pl.ds(off, _L)]
        xi = plsc.load_gather(posx_v, [i])
        yi = plsc.load_gather(posy_v, [i])
        zi = plsc.load_gather(posz_v, [i])
        xj = plsc.load_gather(posx_v, [j])
        yj = plsc.load_gather(posy_v, [j])
        zj = plsc.load_gather(posz_v, [j])
        ti = plsc.load_gather(typ_v, [i])
        tj = plsc.load_gather(typ_v, [j])
        t = ti * N_TYPES + tj
        x0e = plsc.load_gather(x0_v, [t])
        ke = plsc.load_gather(k_v, [t])
        dx = xi - xj
        dy = yi - yj
        dz = zi - zj
        s = dx * dx + dy * dy + dz * dz + jnp.float32(1e-12)
        d = _sqrt16(s)
        diff = d - x0e
        out_v[pl.ds(off, _L)] = ke * diff * diff
        return carry

    lax.fori_loop(0, _CHUNKS, chunk, 0, unroll=8)
    pltpu.sync_copy(out_v, out_h.at[pl.ds(base, _E_PER)])


@functools.partial(
    pl.kernel,
    mesh=plsc.VectorSubcoreMesh(core_axis_name="c", subcore_axis_name="s"),
    out_type=jax.ShapeDtypeStruct((N_BONDS,), jnp.float32),
    compiler_params=pltpu.CompilerParams(needs_layout_passes=False),
    scratch_types=[
        pltpu.VMEM((N_ATOMS,), jnp.float32),
        pltpu.VMEM((N_ATOMS,), jnp.float32),
        pltpu.VMEM((N_ATOMS,), jnp.float32),
        pltpu.VMEM((N_ATOMS,), jnp.int32),
        pltpu.VMEM((N_TYPES * N_TYPES,), jnp.float32),
        pltpu.VMEM((N_TYPES * N_TYPES,), jnp.float32),
        pltpu.VMEM((_E_PER,), jnp.int32),
        pltpu.VMEM((_E_PER,), jnp.int32),
        pltpu.VMEM((_E_PER,), jnp.float32),
        pltpu.SemaphoreType.DMA,
    ],
)
def _harmonic_sc(posx, posy, posz, typ, x0f, kf, src, dst, out,
                 posx_v, posy_v, posz_v, typ_v, x0_v, k_v, src_v, dst_v, out_v,
                 sem):
    _body(posx, posy, posz, typ, x0f, kf, src, dst, out,
          posx_v, posy_v, posz_v, typ_v, x0_v, k_v, src_v, dst_v, out_v, sem)


def kernel(pos, mapping, atom_types, x_0, k_const):
    pos = pos.astype(jnp.float32)
    posx = pos[:, 0]
    posy = pos[:, 1]
    posz = pos[:, 2]
    typ = atom_types.astype(jnp.int32)
    src = mapping[0].astype(jnp.int32)
    dst = mapping[1].astype(jnp.int32)
    x0f = x_0.reshape(-1).astype(jnp.float32)
    kf = k_const.reshape(-1).astype(jnp.float32)
    return _harmonic_sc(posx, posy, posz, typ, x0f, kf, src, dst)


# speedup vs baseline: 1.0936x; 1.0936x over previous
import functools

import jax
import jax.numpy as jnp
from jax import lax
from jax.experimental import pallas as pl
from jax.experimental.pallas import tpu as pltpu
from jax.experimental.pallas import tpu_sc as plsc

N_ATOMS = 10000
N_BONDS = 320000
N_TYPES = 20

_NC = 2
_NS = 16
_NW = _NC * _NS
_L = 16
_E_PER = N_BONDS // _NW
_CHUNKS = _E_PER // _L


def _sqrt16(s):
    i = lax.bitcast_convert_type(s, jnp.int32)
    i = jnp.int32(0x5F3759DF) - lax.shift_right_logical(i, 1)
    r = lax.bitcast_convert_type(i, jnp.float32)
    half = s * jnp.float32(0.5)
    for _ in range(3):
        r = r * (jnp.float32(1.5) - half * r * r)
    return s * r


def _body(pos_h, typ_h, x0_h, k_h, map_h, out_h,
          pos_v, typ_v, x0_v, k_v, src_v, dst_v, out_v, sem):
    wid = lax.axis_index("s") * _NC + lax.axis_index("c")
    base = wid * _E_PER

    copies = [
        pltpu.make_async_copy(pos_h, pos_v, sem),
        pltpu.make_async_copy(typ_h, typ_v, sem),
        pltpu.make_async_copy(x0_h, x0_v, sem),
        pltpu.make_async_copy(k_h, k_v, sem),
        pltpu.make_async_copy(map_h.at[pl.ds(base, _E_PER)], src_v, sem),
        pltpu.make_async_copy(map_h.at[pl.ds(N_BONDS + base, _E_PER)], dst_v, sem),
    ]
    for cp in copies:
        cp.start()
    for cp in copies:
        cp.wait()

    def chunk(c, carry):
        off = c * _L
        i = src_v[pl.ds(off, _L)]
        j = dst_v[pl.ds(off, _L)]
        i3 = i * 3
        j3 = j * 3
        one = jnp.int32(1)
        two = jnp.int32(2)
        xi = plsc.load_gather(pos_v, [i3])
        yi = plsc.load_gather(pos_v, [i3 + one])
        zi = plsc.load_gather(pos_v, [i3 + two])
        xj = plsc.load_gather(pos_v, [j3])
        yj = plsc.load_gather(pos_v, [j3 + one])
        zj = plsc.load_gather(pos_v, [j3 + two])
        ti = plsc.load_gather(typ_v, [i])
        tj = plsc.load_gather(typ_v, [j])
        t = ti * N_TYPES + tj
        x0e = plsc.load_gather(x0_v, [t])
        ke = plsc.load_gather(k_v, [t])
        dx = xi - xj
        dy = yi - yj
        dz = zi - zj
        s = dx * dx + dy * dy + dz * dz + jnp.float32(1e-12)
        d = _sqrt16(s)
        diff = d - x0e
        out_v[pl.ds(off, _L)] = ke * diff * diff
        return carry

    lax.fori_loop(0, _CHUNKS, chunk, 0, unroll=8)
    pltpu.sync_copy(out_v, out_h.at[pl.ds(base, _E_PER)])


@functools.partial(
    pl.kernel,
    mesh=plsc.VectorSubcoreMesh(core_axis_name="c", subcore_axis_name="s"),
    out_type=jax.ShapeDtypeStruct((N_BONDS,), jnp.float32),
    compiler_params=pltpu.CompilerParams(needs_layout_passes=False),
    scratch_types=[
        pltpu.VMEM((N_ATOMS * 3,), jnp.float32),
        pltpu.VMEM((N_ATOMS,), jnp.int32),
        pltpu.VMEM((N_TYPES * N_TYPES,), jnp.float32),
        pltpu.VMEM((N_TYPES * N_TYPES,), jnp.float32),
        pltpu.VMEM((_E_PER,), jnp.int32),
        pltpu.VMEM((_E_PER,), jnp.int32),
        pltpu.VMEM((_E_PER,), jnp.float32),
        pltpu.SemaphoreType.DMA,
    ],
)
def _harmonic_sc(posf, typ, x0f, kf, mapping, out,
                 pos_v, typ_v, x0_v, k_v, src_v, dst_v, out_v, sem):
    _body(posf, typ, x0f, kf, mapping, out,
          pos_v, typ_v, x0_v, k_v, src_v, dst_v, out_v, sem)


def kernel(pos, mapping, atom_types, x_0, k_const):
    posf = pos.astype(jnp.float32).reshape(-1)
    typ = atom_types.astype(jnp.int32)
    mp = mapping.astype(jnp.int32).reshape(-1)
    x0f = x_0.astype(jnp.float32).reshape(-1)
    kf = k_const.astype(jnp.float32).reshape(-1)
    return _harmonic_sc(posf, typ, x0f, kf, mp)


# speedup vs baseline: 1.4943x; 1.3664x over previous
import functools

import jax
import jax.numpy as jnp
from jax import lax
from jax.experimental import pallas as pl
from jax.experimental.pallas import tpu as pltpu
from jax.experimental.pallas import tpu_sc as plsc

N_ATOMS = 10000
N_BONDS = 320000
N_TYPES = 20

_NC = 2
_NS = 16
_NW = _NC * _NS
_L = 16
_E_PER = N_BONDS // _NW
_CHUNKS = _E_PER // _L


def _sqrt16(s):
    i = lax.bitcast_convert_type(s, jnp.int32)
    i = jnp.int32(0x5F3759DF) - lax.shift_right_logical(i, 1)
    r = lax.bitcast_convert_type(i, jnp.float32)
    half = s * jnp.float32(0.5)
    for _ in range(3):
        r = r * (jnp.float32(1.5) - half * r * r)
    return s * r


def _body(pos_h, typ_h, x0_h, k_h, map_h, out_h,
          pos_v, typ_v, x0_v, k_v, src_v, dst_v, out_v, sem):
    wid = lax.axis_index("s") * _NC + lax.axis_index("c")
    base = wid * _E_PER

    copies = [
        pltpu.make_async_copy(pos_h, pos_v, sem),
        pltpu.make_async_copy(typ_h, typ_v, sem),
        pltpu.make_async_copy(x0_h, x0_v, sem),
        pltpu.make_async_copy(k_h, k_v, sem),
        pltpu.make_async_copy(map_h.at[pl.ds(base, _E_PER)], src_v, sem),
        pltpu.make_async_copy(map_h.at[pl.ds(N_BONDS + base, _E_PER)], dst_v, sem),
    ]
    for cp in copies:
        cp.start()
    for cp in copies:
        cp.wait()

    @plsc.parallel_loop(0, _E_PER, step=_L, unroll=8)
    def chunk(off):
        i = src_v[pl.ds(off, _L)]
        j = dst_v[pl.ds(off, _L)]
        i3 = i * 3
        j3 = j * 3
        one = jnp.int32(1)
        two = jnp.int32(2)
        xi = plsc.load_gather(pos_v, [i3])
        yi = plsc.load_gather(pos_v, [i3 + one])
        zi = plsc.load_gather(pos_v, [i3 + two])
        xj = plsc.load_gather(pos_v, [j3])
        yj = plsc.load_gather(pos_v, [j3 + one])
        zj = plsc.load_gather(pos_v, [j3 + two])
        ti = plsc.load_gather(typ_v, [i])
        tj = plsc.load_gather(typ_v, [j])
        t = ti * N_TYPES + tj
        x0e = plsc.load_gather(x0_v, [t])
        ke = plsc.load_gather(k_v, [t])
        dx = xi - xj
        dy = yi - yj
        dz = zi - zj
        s = dx * dx + dy * dy + dz * dz + jnp.float32(1e-12)
        d = _sqrt16(s)
        diff = d - x0e
        out_v[pl.ds(off, _L)] = ke * diff * diff

    pltpu.sync_copy(out_v, out_h.at[pl.ds(base, _E_PER)])


@functools.partial(
    pl.kernel,
    mesh=plsc.VectorSubcoreMesh(core_axis_name="c", subcore_axis_name="s"),
    out_type=jax.ShapeDtypeStruct((N_BONDS,), jnp.float32),
    compiler_params=pltpu.CompilerParams(needs_layout_passes=False),
    scratch_types=[
        pltpu.VMEM((N_ATOMS * 3,), jnp.float32),
        pltpu.VMEM((N_ATOMS,), jnp.int32),
        pltpu.VMEM((N_TYPES * N_TYPES,), jnp.float32),
        pltpu.VMEM((N_TYPES * N_TYPES,), jnp.float32),
        pltpu.VMEM((_E_PER,), jnp.int32),
        pltpu.VMEM((_E_PER,), jnp.int32),
        pltpu.VMEM((_E_PER,), jnp.float32),
        pltpu.SemaphoreType.DMA,
    ],
)
def _harmonic_sc(posf, typ, x0f, kf, mapping, out,
                 pos_v, typ_v, x0_v, k_v, src_v, dst_v, out_v, sem):
    _body(posf, typ, x0f, kf, mapping, out,
          pos_v, typ_v, x0_v, k_v, src_v, dst_v, out_v, sem)


def kernel(pos, mapping, atom_types, x_0, k_const):
    posf = pos.astype(jnp.float32).reshape(-1)
    typ = atom_types.astype(jnp.int32)
    mp = mapping.astype(jnp.int32).reshape(-1)
    x0f = x_0.astype(jnp.float32).reshape(-1)
    kf = k_const.astype(jnp.float32).reshape(-1)
    return _harmonic_sc(posf, typ, x0f, kf, mp)


# speedup vs baseline: 1.6112x; 1.0783x over previous
import functools

import jax
import jax.numpy as jnp
from jax import lax
from jax.experimental import pallas as pl
from jax.experimental.pallas import tpu as pltpu
from jax.experimental.pallas import tpu_sc as plsc

N_ATOMS = 10000
N_BONDS = 320000
N_TYPES = 20

_NC = 2
_NS = 16
_NW = _NC * _NS
_L = 16
_BLK = 128
_NBLK = N_BONDS // _BLK
_BPW = -(-_NBLK // _NW)
_E_PER = _BPW * _BLK
_X0_OFF = N_ATOMS * 3
_K_OFF = _X0_OFF + N_TYPES * N_TYPES
_TAB = _K_OFF + N_TYPES * N_TYPES


def _sqrt16(s):
    i = lax.bitcast_convert_type(s, jnp.int32)
    i = jnp.int32(0x5F3759DF) - lax.shift_right_logical(i, 1)
    r = lax.bitcast_convert_type(i, jnp.float32)
    half = s * jnp.float32(0.5)
    for _ in range(3):
        r = r * (jnp.float32(1.5) - half * r * r)
    return s * r


def _body(tab_h, typ_h, map_h, out_h, tab_v, typ_v, idx_v, out_v, sem):
    wid = lax.axis_index("s") * _NC + lax.axis_index("c")
    ebase = pl.multiple_of((wid * _NBLK) // _NW * _BLK, _BLK)

    copies = [
        pltpu.make_async_copy(tab_h, tab_v, sem),
        pltpu.make_async_copy(typ_h, typ_v, sem),
        pltpu.make_async_copy(map_h.at[:, pl.ds(ebase, _E_PER)], idx_v, sem),
    ]
    for cp in copies:
        cp.start()
    for cp in copies:
        cp.wait()

    @plsc.parallel_loop(0, _E_PER, step=_L, unroll=8)
    def chunk(off):
        i = idx_v[0, pl.ds(off, _L)]
        j = idx_v[1, pl.ds(off, _L)]
        i3 = i * 3
        j3 = j * 3
        one = jnp.int32(1)
        two = jnp.int32(2)
        xi = plsc.load_gather(tab_v, [i3])
        yi = plsc.load_gather(tab_v, [i3 + one])
        zi = plsc.load_gather(tab_v, [i3 + two])
        xj = plsc.load_gather(tab_v, [j3])
        yj = plsc.load_gather(tab_v, [j3 + one])
        zj = plsc.load_gather(tab_v, [j3 + two])
        ti = plsc.load_gather(typ_v, [i])
        tj = plsc.load_gather(typ_v, [j])
        t = ti * N_TYPES + tj
        x0e = plsc.load_gather(tab_v, [t + jnp.int32(_X0_OFF)])
        ke = plsc.load_gather(tab_v, [t + jnp.int32(_K_OFF)])
        dx = xi - xj
        dy = yi - yj
        dz = zi - zj
        s = dx * dx + dy * dy + dz * dz + jnp.float32(1e-12)
        d = _sqrt16(s)
        diff = d - x0e
        out_v[pl.ds(off, _L)] = ke * diff * diff

    pltpu.sync_copy(out_v, out_h.at[pl.ds(ebase, _E_PER)])


@functools.partial(
    pl.kernel,
    mesh=plsc.VectorSubcoreMesh(core_axis_name="c", subcore_axis_name="s"),
    out_type=jax.ShapeDtypeStruct((N_BONDS,), jnp.float32),
    compiler_params=pltpu.CompilerParams(needs_layout_passes=False),
    scratch_types=[
        pltpu.VMEM((_TAB,), jnp.float32),
        pltpu.VMEM((N_ATOMS,), jnp.int32),
        pltpu.VMEM((2, _E_PER), jnp.int32),
        pltpu.VMEM((_E_PER,), jnp.float32),
        pltpu.SemaphoreType.DMA,
    ],
)
def _harmonic_sc(tab, typ, mapping, out, tab_v, typ_v, idx_v, out_v, sem):
    _body(tab, typ, mapping, out, tab_v, typ_v, idx_v, out_v, sem)


def kernel(pos, mapping, atom_types, x_0, k_const):
    tab = jnp.concatenate([
        pos.astype(jnp.float32).reshape(-1),
        x_0.astype(jnp.float32).reshape(-1),
        k_const.astype(jnp.float32).reshape(-1),
    ])
    typ = atom_types.astype(jnp.int32)
    mp = mapping.astype(jnp.int32)
    return _harmonic_sc(tab, typ, mp)


# speedup vs baseline: 2.2441x; 1.3928x over previous
import functools

import jax
import jax.numpy as jnp
from jax import lax
from jax.experimental import pallas as pl
from jax.experimental.pallas import tpu as pltpu
from jax.experimental.pallas import tpu_sc as plsc

N_ATOMS = 10000
N_BONDS = 320000
N_TYPES = 20

_NC = 2
_NS = 16
_NW = _NC * _NS
_L = 16
_BLK = 128
_NBLK = N_BONDS // _BLK
_BPW = -(-_NBLK // _NW)
_E_PER = _BPW * _BLK


def _sqrt16(s):
    i = lax.bitcast_convert_type(s, jnp.int32)
    i = jnp.int32(0x5F3759DF) - lax.shift_right_logical(i, 1)
    r = lax.bitcast_convert_type(i, jnp.float32)
    half = s * jnp.float32(0.5)
    for _ in range(3):
        r = r * (jnp.float32(1.5) - half * r * r)
    return s * r


def _body(pos_h, typ_h, tk_h, map_h, out_h,
          pos_sh, pos_v, typ_v, tk_v, idx_v, out_v, sem):
    wid = lax.axis_index("s") * _NC + lax.axis_index("c")
    ebase = pl.multiple_of((wid * _NBLK) // _NW * _BLK, _BLK)

    copies = [
        pltpu.make_async_copy(typ_h, typ_v, sem),
        pltpu.make_async_copy(tk_h, tk_v, sem),
        pltpu.make_async_copy(map_h.at[:, pl.ds(ebase, _E_PER)], idx_v, sem),
    ]
    for cp in copies:
        cp.start()

    @pl.when(lax.axis_index("s") == 0)
    def _():
        pltpu.sync_copy(pos_h, pos_sh)

    plsc.subcore_barrier()
    pltpu.sync_copy(pos_sh, pos_v)

    for cp in copies:
        cp.wait()

    c0 = jnp.zeros((_L,), jnp.int32)
    c1 = jnp.full((_L,), 1, jnp.int32)
    c2 = jnp.full((_L,), 2, jnp.int32)

    @plsc.parallel_loop(0, _E_PER, step=_L, unroll=8)
    def chunk(off):
        i = idx_v[0, pl.ds(off, _L)]
        j = idx_v[1, pl.ds(off, _L)]
        xi = plsc.load_gather(pos_v, [c0, i])
        yi = plsc.load_gather(pos_v, [c1, i])
        zi = plsc.load_gather(pos_v, [c2, i])
        xj = plsc.load_gather(pos_v, [c0, j])
        yj = plsc.load_gather(pos_v, [c1, j])
        zj = plsc.load_gather(pos_v, [c2, j])
        ti = plsc.load_gather(typ_v, [i])
        tj = plsc.load_gather(typ_v, [j])
        t = ti * N_TYPES + tj
        x0e = plsc.load_gather(tk_v, [t])
        ke = plsc.load_gather(tk_v, [t + jnp.int32(N_TYPES * N_TYPES)])
        dx = xi - xj
        dy = yi - yj
        dz = zi - zj
        s = dx * dx + dy * dy + dz * dz + jnp.float32(1e-12)
        d = _sqrt16(s)
        diff = d - x0e
        out_v[pl.ds(off, _L)] = ke * diff * diff

    pltpu.sync_copy(out_v, out_h.at[pl.ds(ebase, _E_PER)])


@functools.partial(
    pl.kernel,
    mesh=plsc.VectorSubcoreMesh(core_axis_name="c", subcore_axis_name="s"),
    out_type=jax.ShapeDtypeStruct((N_BONDS,), jnp.float32),
    compiler_params=pltpu.CompilerParams(needs_layout_passes=False),
    scratch_types=[
        pltpu.VMEM_SHARED((3, N_ATOMS), jnp.float32),
        pltpu.VMEM((3, N_ATOMS), jnp.float32),
        pltpu.VMEM((N_ATOMS,), jnp.int32),
        pltpu.VMEM((2 * N_TYPES * N_TYPES,), jnp.float32),
        pltpu.VMEM((2, _E_PER), jnp.int32),
        pltpu.VMEM((_E_PER,), jnp.float32),
        pltpu.SemaphoreType.DMA,
    ],
)
def _harmonic_sc(pos_t, typ, tk, mapping, out,
                 pos_sh, pos_v, typ_v, tk_v, idx_v, out_v, sem):
    _body(pos_t, typ, tk, mapping, out,
          pos_sh, pos_v, typ_v, tk_v, idx_v, out_v, sem)


def kernel(pos, mapping, atom_types, x_0, k_const):
    pos_t = pos.astype(jnp.float32).T
    typ = atom_types.astype(jnp.int32)
    mp = mapping.astype(jnp.int32)
    tk = jnp.concatenate([x_0.astype(jnp.float32).reshape(-1),
                          k_const.astype(jnp.float32).reshape(-1)])
    return _harmonic_sc(pos_t, typ, tk, mp)
